# baseline (speedup 1.0000x reference)
import jax
import jax.numpy as jnp
from jax.experimental import pallas as pl
from jax.experimental.pallas import tpu as pltpu

_EPS = 1e-5
_B = 8
_STREAMS = 2

_BF = jnp.bfloat16
_F32 = jnp.float32


def _im2col_dot(act, imcol_ref, wcat_ref):
    b, ho, wo, c = act.shape
    m = b * ho * wo
    zrow = jnp.zeros((b, 1, wo, c), _BF)
    zcol = jnp.zeros((b, ho, 1, c), _BF)
    cols = (jnp.concatenate([zcol, act[:, :, 0:wo - 1]], axis=2),
            act,
            jnp.concatenate([act[:, :, 1:wo], zcol], axis=2))
    for dy in range(3):
        for dx in range(3):
            v = cols[dx]
            if dy == 0:
                v = jnp.concatenate([zrow, v[:, 0:ho - 1]], axis=1)
            elif dy == 2:
                v = jnp.concatenate([v[:, 1:ho], zrow], axis=1)
            blk = (dy * 3 + dx) * c
            imcol_ref[:, :, :, blk:blk + c] = v
    return jnp.dot(imcol_ref[...].reshape(m, 9 * c), wcat_ref[...],
                   preferred_element_type=_F32)


def _stream_net(x2v, o_ref, osl, params, imA_ref, imB_ref, acts_ref, xbuf_ref):
    (s10, h10, wc0, s20, h20, w20, sw,
     s11, h11, w11, s21, h21, w21,
     s12, h12, w12, s22, h22, w22,
     s13, h13, w13, s23, h23, w23) = params
    b, h, wo, c2 = x2v.shape
    ho = h // 2
    cin = c2 // 2
    cout = wc0.shape[-1]
    m = b * ho * wo

    act = jnp.maximum(x2v * s10[...] + h10[...], 0.0)
    act4 = act.reshape(b, ho, 2, wo, c2)
    even = act4[:, :, 0].astype(_BF)
    odd = act4[:, :, 1].astype(_BF)

    acts_ref[...] = even[..., :cin]

    zrow = jnp.zeros((b, 1, wo, c2), _BF)
    zcol = jnp.zeros((b, ho, 1, c2), _BF)
    evenL = jnp.concatenate([zcol, even[:, :, 0:wo - 1]], axis=2)
    oddL = jnp.concatenate([zcol, odd[:, :, 0:wo - 1]], axis=2)
    odd_dn = jnp.concatenate([zrow, odd[:, 0:ho - 1]], axis=1)
    oddL_dn = jnp.concatenate([zrow, oddL[:, 0:ho - 1]], axis=1)
    blocks = (oddL_dn, odd_dn, evenL, even, oddL, odd)
    for kb, v in enumerate(blocks):
        imA_ref[:, :, :, kb * c2:(kb + 1) * c2] = v
    k0 = 6 * c2
    u = jnp.dot(imA_ref[:, :, :, 0:k0].reshape(m, k0), wc0[...],
                preferred_element_type=_F32)

    a2 = jnp.maximum(u.reshape(b, ho, wo, cout) * s20[...] + h20[...],
                     0.0).astype(_BF)
    acc2 = _im2col_dot(a2, imB_ref, w20)
    acc2 = jnp.dot(acts_ref[...].reshape(m, cin), sw[...],
                   preferred_element_type=_F32) + acc2
    xbuf_ref[...] = acc2.reshape(b, ho, wo, cout)

    layers = ((s11, h11, w11, s21, h21, w21),
              (s12, h12, w12, s22, h22, w22),
              (s13, h13, w13, s23, h23, w23))
    for li, (s1, h1, w1, s2, h2, w2) in enumerate(layers):
        p1, p2 = (imA_ref, imB_ref) if li % 2 == 0 else (imB_ref, imA_ref)
        a1 = jnp.maximum(xbuf_ref[...] * s1[...] + h1[...], 0.0).astype(_BF)
        uu = _im2col_dot(a1, p1, w1)
        a2 = jnp.maximum(uu.reshape(b, ho, wo, cout) * s2[...] + h2[...],
                         0.0).astype(_BF)
        vv = _im2col_dot(a2, p2, w2)
        out = vv.reshape(b, ho, wo, cout) + xbuf_ref[...]
        if li == 2:
            o_ref[osl] = out.astype(o_ref.dtype)
        else:
            xbuf_ref[...] = out


def _net_kernel(*refs):
    x2_ref = refs[0]
    params = refs[1:26]
    o_ref = refs[26]
    scr = refs[27:]
    b = x2_ref.shape[0]
    hb = b // _STREAMS
    for s in range(_STREAMS):
        sl = slice(s * hb, (s + 1) * hb)
        _stream_net(x2_ref[sl], o_ref, sl, params,
                    scr[4 * s], scr[4 * s + 1], scr[4 * s + 2], scr[4 * s + 3])


def _fold_bn(gamma, beta, mean, var):
    scale = gamma / jnp.sqrt(var + _EPS)
    shift = beta - mean * scale
    return scale, shift


def kernel(x, l0_bn1_gamma, l0_bn1_beta, l0_bn1_mean, l0_bn1_var, l0_conv1_w, l0_bn2_gamma, l0_bn2_beta, l0_bn2_mean, l0_bn2_var, l0_conv2_w, l0_short_w, l1_bn1_gamma, l1_bn1_beta, l1_bn1_mean, l1_bn1_var, l1_conv1_w, l1_bn2_gamma, l1_bn2_beta, l1_bn2_mean, l1_bn2_var, l1_conv2_w, l2_bn1_gamma, l2_bn1_beta, l2_bn1_mean, l2_bn1_var, l2_conv1_w, l2_bn2_gamma, l2_bn2_beta, l2_bn2_mean, l2_bn2_var, l2_conv2_w, l3_bn1_gamma, l3_bn1_beta, l3_bn1_mean, l3_bn1_var, l3_conv1_w, l3_bn2_gamma, l3_bn2_beta, l3_bn2_mean, l3_bn2_var, l3_conv2_w):
    n, cin, h, wsp = x.shape
    cout = l0_conv1_w.shape[-1]
    ho, wo = h // 2, wsp // 2
    c2 = 2 * cin
    b = _B
    hb = b // _STREAMS

    x2 = jnp.transpose(x.astype(_BF), (0, 2, 3, 1)).reshape(n, h, wo, c2)

    s10, h10 = _fold_bn(l0_bn1_gamma, l0_bn1_beta, l0_bn1_mean, l0_bn1_var)
    s10 = jnp.concatenate([s10, s10]).reshape(1, 1, 1, c2)
    h10 = jnp.concatenate([h10, h10]).reshape(1, 1, 1, c2)
    wc0 = jnp.stack(
        [jnp.concatenate([jnp.zeros_like(l0_conv1_w[:, 0]), l0_conv1_w[:, 0]],
                         axis=1),
         jnp.concatenate([l0_conv1_w[:, 1], l0_conv1_w[:, 2]], axis=1)],
        axis=1).reshape(6 * c2, cout).astype(_BF)

    def vec(s):
        return s.reshape(1, 1, 1, -1)

    def wcat(w):
        return w.reshape(-1, w.shape[-1]).astype(_BF)

    s20, h20 = _fold_bn(l0_bn2_gamma, l0_bn2_beta, l0_bn2_mean, l0_bn2_var)
    s11, h11 = _fold_bn(l1_bn1_gamma, l1_bn1_beta, l1_bn1_mean, l1_bn1_var)
    s21, h21 = _fold_bn(l1_bn2_gamma, l1_bn2_beta, l1_bn2_mean, l1_bn2_var)
    s12, h12 = _fold_bn(l2_bn1_gamma, l2_bn1_beta, l2_bn1_mean, l2_bn1_var)
    s22, h22 = _fold_bn(l2_bn2_gamma, l2_bn2_beta, l2_bn2_mean, l2_bn2_var)
    s13, h13 = _fold_bn(l3_bn1_gamma, l3_bn1_beta, l3_bn1_mean, l3_bn1_var)
    s23, h23 = _fold_bn(l3_bn2_gamma, l3_bn2_beta, l3_bn2_mean, l3_bn2_var)

    args = [
        x2,
        s10, h10, wc0,
        vec(s20), vec(h20), wcat(l0_conv2_w), l0_short_w.astype(_BF),
        vec(s11), vec(h11), wcat(l1_conv1_w),
        vec(s21), vec(h21), wcat(l1_conv2_w),
        vec(s12), vec(h12), wcat(l2_conv1_w),
        vec(s22), vec(h22), wcat(l2_conv2_w),
        vec(s13), vec(h13), wcat(l3_conv1_w),
        vec(s23), vec(h23), wcat(l3_conv2_w),
    ]

    def const(shape):
        nd = len(shape)
        return pl.BlockSpec(shape, lambda bi: (0,) * nd)

    in_specs = [pl.BlockSpec((b, h, wo, c2), lambda bi: (bi, 0, 0, 0))]
    in_specs += [const(a.shape) for a in args[1:]]

    stream_scratch = [
        pltpu.VMEM((hb, ho, wo, 9 * cout), _BF),
        pltpu.VMEM((hb, ho, wo, 9 * cout), _BF),
        pltpu.VMEM((hb, ho, wo, cin), _BF),
        pltpu.VMEM((hb, ho, wo, cout), _F32),
    ]

    out = pl.pallas_call(
        _net_kernel,
        out_shape=jax.ShapeDtypeStruct((n, ho, wo, cout), x.dtype),
        grid=(n // b,),
        in_specs=in_specs,
        out_specs=pl.BlockSpec((b, ho, wo, cout), lambda bi: (bi, 0, 0, 0)),
        scratch_shapes=stream_scratch * _STREAMS,
        compiler_params=pltpu.CompilerParams(
            dimension_semantics=("parallel",),
            vmem_limit_bytes=64 * 1024 * 1024,
        ),
    )(*args)

    return jnp.transpose(out, (0, 3, 1, 2))


# speedup vs baseline: 1.1625x; 1.1625x over previous
import jax
import jax.numpy as jnp
from jax.experimental import pallas as pl
from jax.experimental.pallas import tpu as pltpu

_EPS = 1e-5
_B = 4
_STREAMS = 2

_BF = jnp.bfloat16
_F32 = jnp.float32


def _im2col_dot(act, imcol_ref, wcat_ref):
    b, ho, wo, c = act.shape
    m = b * ho * wo
    zrow = jnp.zeros((b, 1, wo, c), _BF)
    zcol = jnp.zeros((b, ho, 1, c), _BF)
    cols = (jnp.concatenate([zcol, act[:, :, 0:wo - 1]], axis=2),
            act,
            jnp.concatenate([act[:, :, 1:wo], zcol], axis=2))
    for dy in range(3):
        for dx in range(3):
            v = cols[dx]
            if dy == 0:
                v = jnp.concatenate([zrow, v[:, 0:ho - 1]], axis=1)
            elif dy == 2:
                v = jnp.concatenate([v[:, 1:ho], zrow], axis=1)
            blk = (dy * 3 + dx) * c
            imcol_ref[:, :, :, blk:blk + c] = v
    return jnp.dot(imcol_ref[...].reshape(m, 9 * c), wcat_ref[...],
                   preferred_element_type=_F32)


def _stream_net(x2v, o_ref, osl, params, imA_ref, imB_ref, acts_ref, xbuf_ref):
    (s10, h10, wc0, s20, h20, w20, sw,
     s11, h11, w11, s21, h21, w21,
     s12, h12, w12, s22, h22, w22,
     s13, h13, w13, s23, h23, w23) = params
    b, h, wo, c2 = x2v.shape
    ho = h // 2
    cin = c2 // 2
    cout = wc0.shape[-1]
    m = b * ho * wo

    act = jnp.maximum(x2v * s10[...] + h10[...], 0.0)
    act4 = act.reshape(b, ho, 2, wo, c2)
    even = act4[:, :, 0].astype(_BF)
    odd = act4[:, :, 1].astype(_BF)

    acts_ref[...] = even[..., :cin]

    zrow = jnp.zeros((b, 1, wo, c2), _BF)
    zcol = jnp.zeros((b, ho, 1, c2), _BF)
    evenL = jnp.concatenate([zcol, even[:, :, 0:wo - 1]], axis=2)
    oddL = jnp.concatenate([zcol, odd[:, :, 0:wo - 1]], axis=2)
    odd_dn = jnp.concatenate([zrow, odd[:, 0:ho - 1]], axis=1)
    oddL_dn = jnp.concatenate([zrow, oddL[:, 0:ho - 1]], axis=1)
    blocks = (oddL_dn, odd_dn, evenL, even, oddL, odd)
    for kb, v in enumerate(blocks):
        imA_ref[:, :, :, kb * c2:(kb + 1) * c2] = v
    k0 = 6 * c2
    u = jnp.dot(imA_ref[:, :, :, 0:k0].reshape(m, k0), wc0[...],
                preferred_element_type=_F32)

    a2 = jnp.maximum(u.reshape(b, ho, wo, cout) * s20[...] + h20[...],
                     0.0).astype(_BF)
    acc2 = _im2col_dot(a2, imB_ref, w20)
    acc2 = jnp.dot(acts_ref[...].reshape(m, cin), sw[...],
                   preferred_element_type=_F32) + acc2
    xbuf_ref[...] = acc2.reshape(b, ho, wo, cout)

    layers = ((s11, h11, w11, s21, h21, w21),
              (s12, h12, w12, s22, h22, w22),
              (s13, h13, w13, s23, h23, w23))
    for li, (s1, h1, w1, s2, h2, w2) in enumerate(layers):
        p1, p2 = (imA_ref, imB_ref) if li % 2 == 0 else (imB_ref, imA_ref)
        a1 = jnp.maximum(xbuf_ref[...] * s1[...] + h1[...], 0.0).astype(_BF)
        uu = _im2col_dot(a1, p1, w1)
        a2 = jnp.maximum(uu.reshape(b, ho, wo, cout) * s2[...] + h2[...],
                         0.0).astype(_BF)
        vv = _im2col_dot(a2, p2, w2)
        out = vv.reshape(b, ho, wo, cout) + xbuf_ref[...]
        if li == 2:
            o_ref[osl] = out.astype(o_ref.dtype)
        else:
            xbuf_ref[...] = out


def _net_kernel(*refs):
    x2_ref = refs[0]
    params = refs[1:26]
    o_ref = refs[26]
    scr = refs[27:]
    b = x2_ref.shape[0]
    hb = b // _STREAMS
    for s in range(_STREAMS):
        sl = slice(s * hb, (s + 1) * hb)
        _stream_net(x2_ref[sl], o_ref, sl, params,
                    scr[4 * s], scr[4 * s + 1], scr[4 * s + 2], scr[4 * s + 3])


def _fold_bn(gamma, beta, mean, var):
    scale = gamma / jnp.sqrt(var + _EPS)
    shift = beta - mean * scale
    return scale, shift


def kernel(x, l0_bn1_gamma, l0_bn1_beta, l0_bn1_mean, l0_bn1_var, l0_conv1_w, l0_bn2_gamma, l0_bn2_beta, l0_bn2_mean, l0_bn2_var, l0_conv2_w, l0_short_w, l1_bn1_gamma, l1_bn1_beta, l1_bn1_mean, l1_bn1_var, l1_conv1_w, l1_bn2_gamma, l1_bn2_beta, l1_bn2_mean, l1_bn2_var, l1_conv2_w, l2_bn1_gamma, l2_bn1_beta, l2_bn1_mean, l2_bn1_var, l2_conv1_w, l2_bn2_gamma, l2_bn2_beta, l2_bn2_mean, l2_bn2_var, l2_conv2_w, l3_bn1_gamma, l3_bn1_beta, l3_bn1_mean, l3_bn1_var, l3_conv1_w, l3_bn2_gamma, l3_bn2_beta, l3_bn2_mean, l3_bn2_var, l3_conv2_w):
    n, cin, h, wsp = x.shape
    cout = l0_conv1_w.shape[-1]
    ho, wo = h // 2, wsp // 2
    c2 = 2 * cin
    b = _B
    hb = b // _STREAMS

    x2 = jnp.transpose(x.astype(_BF), (0, 2, 3, 1)).reshape(n, h, wo, c2)

    s10, h10 = _fold_bn(l0_bn1_gamma, l0_bn1_beta, l0_bn1_mean, l0_bn1_var)
    s10 = jnp.concatenate([s10, s10]).reshape(1, 1, 1, c2)
    h10 = jnp.concatenate([h10, h10]).reshape(1, 1, 1, c2)
    wc0 = jnp.stack(
        [jnp.concatenate([jnp.zeros_like(l0_conv1_w[:, 0]), l0_conv1_w[:, 0]],
                         axis=1),
         jnp.concatenate([l0_conv1_w[:, 1], l0_conv1_w[:, 2]], axis=1)],
        axis=1).reshape(6 * c2, cout).astype(_BF)

    def vec(s):
        return s.reshape(1, 1, 1, -1)

    def wcat(w):
        return w.reshape(-1, w.shape[-1]).astype(_BF)

    s20, h20 = _fold_bn(l0_bn2_gamma, l0_bn2_beta, l0_bn2_mean, l0_bn2_var)
    s11, h11 = _fold_bn(l1_bn1_gamma, l1_bn1_beta, l1_bn1_mean, l1_bn1_var)
    s21, h21 = _fold_bn(l1_bn2_gamma, l1_bn2_beta, l1_bn2_mean, l1_bn2_var)
    s12, h12 = _fold_bn(l2_bn1_gamma, l2_bn1_beta, l2_bn1_mean, l2_bn1_var)
    s22, h22 = _fold_bn(l2_bn2_gamma, l2_bn2_beta, l2_bn2_mean, l2_bn2_var)
    s13, h13 = _fold_bn(l3_bn1_gamma, l3_bn1_beta, l3_bn1_mean, l3_bn1_var)
    s23, h23 = _fold_bn(l3_bn2_gamma, l3_bn2_beta, l3_bn2_mean, l3_bn2_var)

    args = [
        x2,
        s10, h10, wc0,
        vec(s20), vec(h20), wcat(l0_conv2_w), l0_short_w.astype(_BF),
        vec(s11), vec(h11), wcat(l1_conv1_w),
        vec(s21), vec(h21), wcat(l1_conv2_w),
        vec(s12), vec(h12), wcat(l2_conv1_w),
        vec(s22), vec(h22), wcat(l2_conv2_w),
        vec(s13), vec(h13), wcat(l3_conv1_w),
        vec(s23), vec(h23), wcat(l3_conv2_w),
    ]

    def const(shape):
        nd = len(shape)
        return pl.BlockSpec(shape, lambda bi: (0,) * nd)

    in_specs = [pl.BlockSpec((b, h, wo, c2), lambda bi: (bi, 0, 0, 0))]
    in_specs += [const(a.shape) for a in args[1:]]

    stream_scratch = [
        pltpu.VMEM((hb, ho, wo, 9 * cout), _BF),
        pltpu.VMEM((hb, ho, wo, 9 * cout), _BF),
        pltpu.VMEM((hb, ho, wo, cin), _BF),
        pltpu.VMEM((hb, ho, wo, cout), _F32),
    ]

    out = pl.pallas_call(
        _net_kernel,
        out_shape=jax.ShapeDtypeStruct((n, ho, wo, cout), x.dtype),
        grid=(n // b,),
        in_specs=in_specs,
        out_specs=pl.BlockSpec((b, ho, wo, cout), lambda bi: (bi, 0, 0, 0)),
        scratch_shapes=stream_scratch * _STREAMS,
        compiler_params=pltpu.CompilerParams(
            dimension_semantics=("parallel",),
            vmem_limit_bytes=64 * 1024 * 1024,
        ),
    )(*args)

    return jnp.transpose(out, (0, 3, 1, 2))


# speedup vs baseline: 1.1869x; 1.0210x over previous
import jax
import jax.numpy as jnp
from jax.experimental import pallas as pl
from jax.experimental.pallas import tpu as pltpu

_EPS = 1e-5
_B = 4
_STREAMS = 2

_BF = jnp.bfloat16
_F32 = jnp.float32


def _im2col_dot(act, imcol_ref, wcat_ref):
    b, ho, wo, c = act.shape
    m = b * ho * wo
    zrow = jnp.zeros((b, 1, wo, c), _BF)
    zcol = jnp.zeros((b, ho, 1, c), _BF)
    cols = (jnp.concatenate([zcol, act[:, :, 0:wo - 1]], axis=2),
            act,
            jnp.concatenate([act[:, :, 1:wo], zcol], axis=2))
    for dy in range(3):
        for dx in range(3):
            v = cols[dx]
            if dy == 0:
                v = jnp.concatenate([zrow, v[:, 0:ho - 1]], axis=1)
            elif dy == 2:
                v = jnp.concatenate([v[:, 1:ho], zrow], axis=1)
            blk = (dy * 3 + dx) * c
            imcol_ref[:, :, :, blk:blk + c] = v
    return jnp.dot(imcol_ref[...].reshape(m, 9 * c), wcat_ref[...],
                   preferred_element_type=_F32)


def _stream_net(x2v, o_ref, osl, params, imA_ref, imB_ref, acts_ref, xbuf_ref):
    (s10, h10, wc0, s20, h20, w20, sw,
     s11, h11, w11, s21, h21, w21,
     s12, h12, w12, s22, h22, w22,
     s13, h13, w13, s23, h23, w23) = params
    b, h, wo, c2 = x2v.shape
    ho = h // 2
    cin = c2 // 2
    cout = wc0.shape[-1]
    m = b * ho * wo

    act = jnp.maximum(x2v * s10[...] + h10[...], 0.0)
    act4 = act.reshape(b, ho, 2, wo, c2)
    even = act4[:, :, 0].astype(_BF)
    odd = act4[:, :, 1].astype(_BF)

    acts_ref[...] = even[..., :cin]

    zrow = jnp.zeros((b, 1, wo, c2), _BF)
    zcol = jnp.zeros((b, ho, 1, c2), _BF)
    evenL = jnp.concatenate([zcol, even[:, :, 0:wo - 1]], axis=2)
    oddL = jnp.concatenate([zcol, odd[:, :, 0:wo - 1]], axis=2)
    odd_dn = jnp.concatenate([zrow, odd[:, 0:ho - 1]], axis=1)
    oddL_dn = jnp.concatenate([zrow, oddL[:, 0:ho - 1]], axis=1)
    blocks = (oddL_dn, odd_dn, evenL, even, oddL, odd)
    for kb, v in enumerate(blocks):
        imA_ref[:, :, :, kb * c2:(kb + 1) * c2] = v
    k0 = 6 * c2
    u = jnp.dot(imA_ref[:, :, :, 0:k0].reshape(m, k0), wc0[...],
                preferred_element_type=_F32)

    a2 = jnp.maximum(u.reshape(b, ho, wo, cout) * s20[...] + h20[...],
                     0.0).astype(_BF)
    acc2 = _im2col_dot(a2, imB_ref, w20)
    acc2 = jnp.dot(acts_ref[...].reshape(m, cin), sw[...],
                   preferred_element_type=_F32) + acc2
    xbuf_ref[...] = acc2.reshape(b, ho, wo, cout)

    layers = ((s11, h11, w11, s21, h21, w21),
              (s12, h12, w12, s22, h22, w22),
              (s13, h13, w13, s23, h23, w23))
    for li, (s1, h1, w1, s2, h2, w2) in enumerate(layers):
        p1, p2 = (imA_ref, imB_ref) if li % 2 == 0 else (imB_ref, imA_ref)
        a1 = jnp.maximum(xbuf_ref[...] * s1[...] + h1[...], 0.0).astype(_BF)
        uu = _im2col_dot(a1, p1, w1)
        a2 = jnp.maximum(uu.reshape(b, ho, wo, cout) * s2[...] + h2[...],
                         0.0).astype(_BF)
        vv = _im2col_dot(a2, p2, w2)
        out = vv.reshape(b, ho, wo, cout) + xbuf_ref[...]
        if li == 2:
            o_ref[osl] = out.astype(o_ref.dtype)
        else:
            xbuf_ref[...] = out


def _net_kernel(*refs):
    x2_ref = refs[0]
    params = refs[1:26]
    o_ref = refs[26]
    scr = refs[27:]
    b = x2_ref.shape[0]
    hb = b // _STREAMS
    for s in range(_STREAMS):
        sl = slice(s * hb, (s + 1) * hb)
        _stream_net(x2_ref[sl], o_ref, sl, params,
                    scr[4 * s], scr[4 * s + 1], scr[4 * s + 2], scr[4 * s + 3])


def _fold_bn(gamma, beta, mean, var):
    scale = gamma / jnp.sqrt(var + _EPS)
    shift = beta - mean * scale
    return scale, shift


def kernel(x, l0_bn1_gamma, l0_bn1_beta, l0_bn1_mean, l0_bn1_var, l0_conv1_w, l0_bn2_gamma, l0_bn2_beta, l0_bn2_mean, l0_bn2_var, l0_conv2_w, l0_short_w, l1_bn1_gamma, l1_bn1_beta, l1_bn1_mean, l1_bn1_var, l1_conv1_w, l1_bn2_gamma, l1_bn2_beta, l1_bn2_mean, l1_bn2_var, l1_conv2_w, l2_bn1_gamma, l2_bn1_beta, l2_bn1_mean, l2_bn1_var, l2_conv1_w, l2_bn2_gamma, l2_bn2_beta, l2_bn2_mean, l2_bn2_var, l2_conv2_w, l3_bn1_gamma, l3_bn1_beta, l3_bn1_mean, l3_bn1_var, l3_conv1_w, l3_bn2_gamma, l3_bn2_beta, l3_bn2_mean, l3_bn2_var, l3_conv2_w):
    n, cin, h, wsp = x.shape
    cout = l0_conv1_w.shape[-1]
    ho, wo = h // 2, wsp // 2
    c2 = 2 * cin
    b = _B
    hb = b // _STREAMS

    x2 = jnp.transpose(x, (0, 2, 3, 1)).reshape(n, h, wo, c2)

    s10, h10 = _fold_bn(l0_bn1_gamma, l0_bn1_beta, l0_bn1_mean, l0_bn1_var)
    s10 = jnp.concatenate([s10, s10]).reshape(1, 1, 1, c2)
    h10 = jnp.concatenate([h10, h10]).reshape(1, 1, 1, c2)
    wc0 = jnp.stack(
        [jnp.concatenate([jnp.zeros_like(l0_conv1_w[:, 0]), l0_conv1_w[:, 0]],
                         axis=1),
         jnp.concatenate([l0_conv1_w[:, 1], l0_conv1_w[:, 2]], axis=1)],
        axis=1).reshape(6 * c2, cout).astype(_BF)

    def vec(s):
        return s.reshape(1, 1, 1, -1)

    def wcat(w):
        return w.reshape(-1, w.shape[-1]).astype(_BF)

    s20, h20 = _fold_bn(l0_bn2_gamma, l0_bn2_beta, l0_bn2_mean, l0_bn2_var)
    s11, h11 = _fold_bn(l1_bn1_gamma, l1_bn1_beta, l1_bn1_mean, l1_bn1_var)
    s21, h21 = _fold_bn(l1_bn2_gamma, l1_bn2_beta, l1_bn2_mean, l1_bn2_var)
    s12, h12 = _fold_bn(l2_bn1_gamma, l2_bn1_beta, l2_bn1_mean, l2_bn1_var)
    s22, h22 = _fold_bn(l2_bn2_gamma, l2_bn2_beta, l2_bn2_mean, l2_bn2_var)
    s13, h13 = _fold_bn(l3_bn1_gamma, l3_bn1_beta, l3_bn1_mean, l3_bn1_var)
    s23, h23 = _fold_bn(l3_bn2_gamma, l3_bn2_beta, l3_bn2_mean, l3_bn2_var)

    args = [
        x2,
        s10, h10, wc0,
        vec(s20), vec(h20), wcat(l0_conv2_w), l0_short_w.astype(_BF),
        vec(s11), vec(h11), wcat(l1_conv1_w),
        vec(s21), vec(h21), wcat(l1_conv2_w),
        vec(s12), vec(h12), wcat(l2_conv1_w),
        vec(s22), vec(h22), wcat(l2_conv2_w),
        vec(s13), vec(h13), wcat(l3_conv1_w),
        vec(s23), vec(h23), wcat(l3_conv2_w),
    ]

    def const(shape):
        nd = len(shape)
        return pl.BlockSpec(shape, lambda bi: (0,) * nd)

    in_specs = [pl.BlockSpec((b, h, wo, c2), lambda bi: (bi, 0, 0, 0))]
    in_specs += [const(a.shape) for a in args[1:]]

    stream_scratch = [
        pltpu.VMEM((hb, ho, wo, 9 * cout), _BF),
        pltpu.VMEM((hb, ho, wo, 9 * cout), _BF),
        pltpu.VMEM((hb, ho, wo, cin), _BF),
        pltpu.VMEM((hb, ho, wo, cout), _F32),
    ]

    out = pl.pallas_call(
        _net_kernel,
        out_shape=jax.ShapeDtypeStruct((n, ho, wo, cout), x.dtype),
        grid=(n // b,),
        in_specs=in_specs,
        out_specs=pl.BlockSpec((b, ho, wo, cout), lambda bi: (bi, 0, 0, 0)),
        scratch_shapes=stream_scratch * _STREAMS,
        compiler_params=pltpu.CompilerParams(
            dimension_semantics=("parallel",),
            vmem_limit_bytes=64 * 1024 * 1024,
        ),
    )(*args)

    return jnp.transpose(out, (0, 3, 1, 2))
